# baseline (speedup 1.0000x reference)
import functools

import jax
import jax.numpy as jnp
from jax import lax
from jax.experimental import pallas as pl
from jax.experimental.pallas import tpu as pltpu
from jax.experimental.pallas import tpu_sc as plsc

DIM = 512
KQ = 48000
TEMP = 0.007
TOPN = 32
RK = 4
BSRC = 64
BTGT = 256
BT2 = BTGT // 2
NB = 15
LBLK = KQ // NB
CHUNK = 128
CPB = LBLK // CHUNK
NCH = KQ // CHUNK
CW = TOPN * CHUNK
BIG = 1 << 30


def _sim_body(feats_ref, tf0_ref, q_ref, sim_ref, sim0_ref, logz_ref,
              fidx_ref, cidx_ref, tn_ref, cm_ref, m_ref, s_ref):
    h = pl.program_id(0)
    k = pl.program_id(1)

    @pl.when(k == 0)
    def _init():
        t = feats_ref[pl.ds(BSRC + h * BT2, BT2), :]
        t0 = tf0_ref[...]
        tn_ref[0:BT2, :] = t / (jnp.sqrt(jnp.sum(t * t, axis=1, keepdims=True)) + 1e-12)
        tn_ref[BT2:, :] = t0 / (jnp.sqrt(jnp.sum(t0 * t0, axis=1, keepdims=True)) + 1e-12)
        m_ref[...] = jnp.full((BT2, 1), -jnp.inf, jnp.float32)
        s_ref[...] = jnp.zeros((BT2, 1), jnp.float32)

    qblk = q_ref[...]
    src = feats_ref[0:BSRC, :]
    head = jnp.where(k == 0, src, qblk[0:BSRC, :])
    q_eff = jnp.concatenate([head, qblk[BSRC:, :]], axis=0)
    qn = q_eff / (jnp.sqrt(jnp.sum(q_eff * q_eff, axis=1, keepdims=True)) + 1e-12)
    s_all = lax.dot_general(tn_ref[...], qn, (((1,), (1,)), ((), ())),
                            preferred_element_type=jnp.float32)
    sim_b = s_all[0:BT2, :]
    sim0_b = s_all[BT2:, :]
    sim_ref[...] = sim_b
    sim0_ref[...] = sim0_b
    cm_ref[pl.ds(k, 1)] = jnp.max(
        sim_b.reshape(BT2, CPB, CHUNK), axis=2).reshape(1, BT2, CPB)

    logits = sim_b / TEMP
    bm = jnp.max(logits, axis=1, keepdims=True)
    new_m = jnp.maximum(m_ref[...], bm)
    s_ref[...] = s_ref[...] * jnp.exp(m_ref[...] - new_m) + \
        jnp.sum(jnp.exp(logits - new_m), axis=1, keepdims=True)
    m_ref[...] = new_m

    @pl.when(k == NB - 1)
    def _fin():
        logz_ref[...] = m_ref[...] + jnp.log(s_ref[...])
        cm0 = jnp.concatenate([cm_ref[kk] for kk in range(NB)], axis=1)
        col = lax.broadcasted_iota(jnp.int32, (BT2, NCH), 1)
        row = lax.broadcasted_iota(jnp.int32, (BT2, 1), 0) + h * BT2
        ncol = lax.broadcasted_iota(jnp.int32, (BT2, TOPN), 1)

        def step(n, carry):
            cm, acc_f, acc_c = carry
            mx = jnp.max(cm, axis=1, keepdims=True)
            idx = jnp.min(jnp.where(cm == mx, col, BIG), axis=1, keepdims=True)
            cm = jnp.where(col == idx, -jnp.inf, cm)
            acc_c = jnp.where(ncol == n, idx, acc_c)
            acc_f = jnp.where(ncol == n, row * NCH + idx, acc_f)
            return cm, acc_f, acc_c

        zero = jnp.zeros((BT2, TOPN), jnp.int32)
        _, acc_f, acc_c = lax.fori_loop(0, TOPN, step, (cm0, zero, zero))
        fidx_ref[...] = acc_f
        cidx_ref[...] = acc_c


def _gather_sc(sim2d, sim02d, lab2d, fidx, cidx):
    info = plsc.get_sparse_core_info()
    nw = info.num_cores * info.num_subcores
    nr = BTGT * TOPN
    bw = nr // nw
    half = bw // 2
    mesh = plsc.VectorSubcoreMesh(core_axis_name="c", subcore_axis_name="s")

    @functools.partial(
        pl.kernel,
        out_type=[
            jax.ShapeDtypeStruct((nr, CHUNK), jnp.float32),
            jax.ShapeDtypeStruct((nr, CHUNK), jnp.float32),
            jax.ShapeDtypeStruct((nr, CHUNK), jnp.int32),
        ],
        mesh=mesh,
        scratch_types=[
            pltpu.VMEM((half,), jnp.int32),
            pltpu.VMEM((half,), jnp.int32),
            pltpu.VMEM((half,), jnp.int32),
            pltpu.VMEM((half,), jnp.int32),
            pltpu.VMEM((half, CHUNK), jnp.float32),
            pltpu.VMEM((half, CHUNK), jnp.float32),
            pltpu.VMEM((half, CHUNK), jnp.float32),
            pltpu.VMEM((half, CHUNK), jnp.float32),
            pltpu.VMEM((half, CHUNK), jnp.int32),
            pltpu.VMEM((half, CHUNK), jnp.int32),
            pltpu.SemaphoreType.DMA,
            pltpu.SemaphoreType.DMA,
        ],
    )
    def gather_kernel(sim_hbm, sim0_hbm, lab_hbm, fidx_hbm, cidx_hbm,
                      cand_hbm, cand0_hbm, clab_hbm,
                      idx_a, idx_b, cidx_a, cidx_b,
                      buf_sa, buf_sb, buf_0a, buf_0b, buf_la, buf_lb,
                      gsem, wsem):
        wid = lax.axis_index("s") * info.num_cores + lax.axis_index("c")
        base = wid * bw
        pltpu.sync_copy(fidx_hbm.at[pl.ds(base, half)], idx_a)
        pltpu.sync_copy(fidx_hbm.at[pl.ds(base + half, half)], idx_b)
        pltpu.sync_copy(cidx_hbm.at[pl.ds(base, half)], cidx_a)
        pltpu.sync_copy(cidx_hbm.at[pl.ds(base + half, half)], cidx_b)
        gathers = [
            (sim_hbm, idx_a, buf_sa, cand_hbm, base),
            (sim_hbm, idx_b, buf_sb, cand_hbm, base + half),
            (sim0_hbm, idx_a, buf_0a, cand0_hbm, base),
            (sim0_hbm, idx_b, buf_0b, cand0_hbm, base + half),
            (lab_hbm, cidx_a, buf_la, clab_hbm, base),
            (lab_hbm, cidx_b, buf_lb, clab_hbm, base + half),
        ]
        copies = [pltpu.async_copy(tbl.at[idx], buf, gsem)
                  for tbl, idx, buf, _, _ in gathers]
        writes = []
        for cp, (_, _, buf, out, off) in zip(copies, gathers):
            cp.wait()
            writes.append(pltpu.async_copy(buf, out.at[pl.ds(off, half)], wsem))
        for wr in writes:
            wr.wait()

    return gather_kernel(sim2d, sim02d, lab2d, fidx, cidx)


def _loss_body(cand_ref, cand0_ref, clab_ref, gcol_ref, logz_ref, srcl_ref,
               tgt_ref, lsum_ref, nc_ref):
    cand = cand_ref[...]
    col = lax.broadcasted_iota(jnp.int32, (BT2, CW), 1)

    def mask_step(_, v):
        m = jnp.max(v, axis=1, keepdims=True)
        return jnp.where(v == m, -jnp.inf, v)

    v = lax.fori_loop(0, TOPN - 1, mask_step, cand)
    theta = jnp.max(v, axis=1, keepdims=True)
    rank = jnp.where(cand >= theta, cand + cand0_ref[...], -jnp.inf)

    logz = logz_ref[...]
    vsum = jnp.zeros((BT2, 1), jnp.float32)
    g0 = jnp.zeros((BT2, 1), jnp.int32)
    lab_q = jnp.zeros((BT2, 1), jnp.int32)
    for n in range(RK):
        m = jnp.max(rank, axis=1, keepdims=True)
        eq = rank == m
        pos = jnp.min(jnp.where(eq, col, BIG), axis=1, keepdims=True)
        sel = col == pos
        val = jnp.sum(jnp.where(sel, cand, 0.0), axis=1, keepdims=True)
        vsum = vsum + (val / TEMP - logz)
        if n == 0:
            g0 = jnp.sum(jnp.where(sel, gcol_ref[...], 0), axis=1, keepdims=True)
            lab_q = jnp.sum(jnp.where(sel, clab_ref[...], 0), axis=1, keepdims=True)
        rank = jnp.where(sel, -jnp.inf, rank)

    part = -(jnp.sum(vsum, axis=0, keepdims=True) / (BTGT * RK))
    lsum_ref[...] = jnp.broadcast_to(part.reshape(1, 1, 1), (1, 1, CHUNK))

    iota64 = lax.broadcasted_iota(jnp.int32, (BT2, BSRC), 1)
    onehot = jnp.where(iota64 == g0, 1.0, 0.0)
    src_val = lax.dot_general(onehot, srcl_ref[...], (((1,), (0,)), ((), ())),
                              preferred_element_type=jnp.float32)
    pred = jnp.where(g0 < BSRC, src_val.astype(jnp.int32), lab_q)
    ncp = jnp.sum(jnp.where(pred == tgt_ref[...], 1, 0), axis=0, keepdims=True)
    nc_ref[...] = jnp.broadcast_to(ncp.reshape(1, 1, 1), (1, 1, CHUNK))


def kernel(features, target_fearures_0, source_labels, target_labels, queue,
           queue_labels):
    f32 = jnp.float32
    sim, sim0, logz, fidx, cidx = pl.pallas_call(
        _sim_body,
        grid=(2, NB),
        in_specs=[
            pl.BlockSpec((BSRC + BTGT, DIM), lambda h, k: (0, 0)),
            pl.BlockSpec((BT2, DIM), lambda h, k: (h, 0)),
            pl.BlockSpec((LBLK, DIM), lambda h, k: (k, 0)),
        ],
        out_specs=[
            pl.BlockSpec((BT2, LBLK), lambda h, k: (h, k)),
            pl.BlockSpec((BT2, LBLK), lambda h, k: (h, k)),
            pl.BlockSpec((BT2, 1), lambda h, k: (h, 0)),
            pl.BlockSpec((BT2, TOPN), lambda h, k: (h, 0)),
            pl.BlockSpec((BT2, TOPN), lambda h, k: (h, 0)),
        ],
        out_shape=[
            jax.ShapeDtypeStruct((BTGT, KQ), f32),
            jax.ShapeDtypeStruct((BTGT, KQ), f32),
            jax.ShapeDtypeStruct((BTGT, 1), f32),
            jax.ShapeDtypeStruct((BTGT, TOPN), jnp.int32),
            jax.ShapeDtypeStruct((BTGT, TOPN), jnp.int32),
        ],
        scratch_shapes=[
            pltpu.VMEM((BT2 + BT2, DIM), f32),
            pltpu.VMEM((NB, BT2, CPB), f32),
            pltpu.VMEM((BT2, 1), f32),
            pltpu.VMEM((BT2, 1), f32),
        ],
        compiler_params=pltpu.CompilerParams(
            dimension_semantics=("parallel", "arbitrary")),
    )(features, target_fearures_0, queue)

    cand, cand0, clab = _gather_sc(
        sim.reshape(BTGT * NCH, CHUNK),
        sim0.reshape(BTGT * NCH, CHUNK),
        queue_labels.reshape(NCH, CHUNK),
        fidx.reshape(BTGT * TOPN),
        cidx.reshape(BTGT * TOPN),
    )

    gcol = (cidx.reshape(BTGT, TOPN, 1) * CHUNK +
            jnp.arange(CHUNK, dtype=jnp.int32).reshape(1, 1, CHUNK)
            ).reshape(BTGT, CW)

    lsum2, nc2 = pl.pallas_call(
        _loss_body,
        grid=(2,),
        in_specs=[
            pl.BlockSpec((BT2, CW), lambda g: (g, 0)),
            pl.BlockSpec((BT2, CW), lambda g: (g, 0)),
            pl.BlockSpec((BT2, CW), lambda g: (g, 0)),
            pl.BlockSpec((BT2, CW), lambda g: (g, 0)),
            pl.BlockSpec((BT2, 1), lambda g: (g, 0)),
            pl.BlockSpec((BSRC, 1), lambda g: (0, 0)),
            pl.BlockSpec((BT2, 1), lambda g: (g, 0)),
        ],
        out_specs=[
            pl.BlockSpec((1, 1, CHUNK), lambda g: (g, 0, 0)),
            pl.BlockSpec((1, 1, CHUNK), lambda g: (g, 0, 0)),
        ],
        out_shape=[
            jax.ShapeDtypeStruct((2, 1, CHUNK), f32),
            jax.ShapeDtypeStruct((2, 1, CHUNK), jnp.int32),
        ],
        compiler_params=pltpu.CompilerParams(
            dimension_semantics=("parallel",)),
    )(
        cand.reshape(BTGT, CW),
        cand0.reshape(BTGT, CW),
        clab.reshape(BTGT, CW),
        gcol,
        logz,
        source_labels.astype(f32).reshape(BSRC, 1),
        target_labels.astype(jnp.int32).reshape(BTGT, 1),
    )
    return (lsum2[0, 0, 0] + lsum2[1, 0, 0], nc2[0, 0, 0] + nc2[1, 0, 0])


# speedup vs baseline: 1.1830x; 1.1830x over previous
import functools

import jax
import jax.numpy as jnp
from jax import lax
from jax.experimental import pallas as pl
from jax.experimental.pallas import tpu as pltpu
from jax.experimental.pallas import tpu_sc as plsc

DIM = 512
KQ = 48000
TEMP = 0.007
TOPN = 32
RK = 4
BSRC = 64
BTGT = 256
NB = 15
LBLK = KQ // NB
CHUNK = 128
CPB = LBLK // CHUNK
NCH = KQ // CHUNK
CW = TOPN * CHUNK
BIG = 1 << 30


def _sim_body(feats_ref, tf0_ref, q_ref, sim_ref, sim0_ref, logz_ref,
              fidx_ref, cidx_ref, tn_ref, cm_ref, m_ref, s_ref):
    k = pl.program_id(0)

    @pl.when(k == 0)
    def _init():
        t = feats_ref[BSRC:, :]
        t0 = tf0_ref[...]
        tn_ref[0:BTGT, :] = t / (jnp.sqrt(jnp.sum(t * t, axis=1, keepdims=True)) + 1e-12) / TEMP
        tn_ref[BTGT:, :] = t0 / (jnp.sqrt(jnp.sum(t0 * t0, axis=1, keepdims=True)) + 1e-12) / TEMP
        m_ref[...] = jnp.full((BTGT, 1), -jnp.inf, jnp.float32)
        s_ref[...] = jnp.zeros((BTGT, 1), jnp.float32)

    qblk = q_ref[...]
    src = feats_ref[0:BSRC, :]
    head = jnp.where(k == 0, src, qblk[0:BSRC, :])
    q_eff = jnp.concatenate([head, qblk[BSRC:, :]], axis=0)
    qn = q_eff / (jnp.sqrt(jnp.sum(q_eff * q_eff, axis=1, keepdims=True)) + 1e-12)
    s_all = lax.dot_general(tn_ref[...], qn, (((1,), (1,)), ((), ())),
                            preferred_element_type=jnp.float32)
    sim_b = s_all[0:BTGT, :]
    sim0_b = s_all[BTGT:, :]
    sim_ref[...] = sim_b
    sim0_ref[...] = sim0_b
    cmb = jnp.max(sim_b.reshape(BTGT, CPB, CHUNK), axis=2)
    cm_ref[pl.ds(k, 1)] = cmb.reshape(1, BTGT, CPB)

    bm = jnp.max(cmb, axis=1, keepdims=True)
    new_m = jnp.maximum(m_ref[...], bm)
    s_ref[...] = s_ref[...] * jnp.exp(m_ref[...] - new_m) + \
        jnp.sum(jnp.exp(sim_b - new_m), axis=1, keepdims=True)
    m_ref[...] = new_m

    @pl.when(k == NB - 1)
    def _fin():
        logz_ref[...] = m_ref[...] + jnp.log(s_ref[...])
        cm0 = jnp.concatenate([cm_ref[kk] for kk in range(NB)], axis=1)
        col = lax.broadcasted_iota(jnp.int32, (BTGT, NCH), 1)
        row = lax.broadcasted_iota(jnp.int32, (BTGT, 1), 0)
        ncol = lax.broadcasted_iota(jnp.int32, (BTGT, TOPN), 1)

        def step(n, carry):
            cm, acc_f, acc_c = carry
            mx = jnp.max(cm, axis=1, keepdims=True)
            idx = jnp.min(jnp.where(cm == mx, col, BIG), axis=1, keepdims=True)
            cm = jnp.where(col == idx, -jnp.inf, cm)
            acc_c = jnp.where(ncol == n, idx, acc_c)
            acc_f = jnp.where(ncol == n, row * NCH + idx, acc_f)
            return cm, acc_f, acc_c

        zero = jnp.zeros((BTGT, TOPN), jnp.int32)
        _, acc_f, acc_c = lax.fori_loop(0, TOPN, step, (cm0, zero, zero))
        fidx_ref[...] = acc_f
        cidx_ref[...] = acc_c


def _gather_sc(sim2d, sim02d, lab2d, fidx, cidx):
    info = plsc.get_sparse_core_info()
    nw = info.num_cores * info.num_subcores
    nr = BTGT * TOPN
    bw = nr // nw
    half = bw // 2
    mesh = plsc.VectorSubcoreMesh(core_axis_name="c", subcore_axis_name="s")

    @functools.partial(
        pl.kernel,
        out_type=[
            jax.ShapeDtypeStruct((nr, CHUNK), jnp.float32),
            jax.ShapeDtypeStruct((nr, CHUNK), jnp.float32),
            jax.ShapeDtypeStruct((nr, CHUNK), jnp.int32),
        ],
        mesh=mesh,
        scratch_types=[
            pltpu.VMEM((half,), jnp.int32),
            pltpu.VMEM((half,), jnp.int32),
            pltpu.VMEM((half,), jnp.int32),
            pltpu.VMEM((half,), jnp.int32),
            pltpu.VMEM((half, CHUNK), jnp.float32),
            pltpu.VMEM((half, CHUNK), jnp.float32),
            pltpu.VMEM((half, CHUNK), jnp.float32),
            pltpu.VMEM((half, CHUNK), jnp.float32),
            pltpu.VMEM((half, CHUNK), jnp.int32),
            pltpu.VMEM((half, CHUNK), jnp.int32),
            pltpu.SemaphoreType.DMA,
            pltpu.SemaphoreType.DMA,
        ],
    )
    def gather_kernel(sim_hbm, sim0_hbm, lab_hbm, fidx_hbm, cidx_hbm,
                      cand_hbm, cand0_hbm, clab_hbm,
                      idx_a, idx_b, cidx_a, cidx_b,
                      buf_sa, buf_sb, buf_0a, buf_0b, buf_la, buf_lb,
                      gsem, wsem):
        wid = lax.axis_index("s") * info.num_cores + lax.axis_index("c")
        base = wid * bw
        pltpu.sync_copy(fidx_hbm.at[pl.ds(base, half)], idx_a)
        pltpu.sync_copy(fidx_hbm.at[pl.ds(base + half, half)], idx_b)
        pltpu.sync_copy(cidx_hbm.at[pl.ds(base, half)], cidx_a)
        pltpu.sync_copy(cidx_hbm.at[pl.ds(base + half, half)], cidx_b)
        gathers = [
            (sim_hbm, idx_a, buf_sa, cand_hbm, base),
            (sim_hbm, idx_b, buf_sb, cand_hbm, base + half),
            (sim0_hbm, idx_a, buf_0a, cand0_hbm, base),
            (sim0_hbm, idx_b, buf_0b, cand0_hbm, base + half),
            (lab_hbm, cidx_a, buf_la, clab_hbm, base),
            (lab_hbm, cidx_b, buf_lb, clab_hbm, base + half),
        ]
        copies = [pltpu.async_copy(tbl.at[idx], buf, gsem)
                  for tbl, idx, buf, _, _ in gathers]
        writes = []
        for cp, (_, _, buf, out, off) in zip(copies, gathers):
            cp.wait()
            writes.append(pltpu.async_copy(buf, out.at[pl.ds(off, half)], wsem))
        for wr in writes:
            wr.wait()

    return gather_kernel(sim2d, sim02d, lab2d, fidx, cidx)


def _loss_body(cand_ref, cand0_ref, clab_ref, gcol_ref, logz_ref, srcl_ref,
               tgt_ref, lsum_ref, nc_ref):
    cand = cand_ref[...]
    col = lax.broadcasted_iota(jnp.int32, (BTGT, CW), 1)

    def mask_step(_, v):
        m = jnp.max(v, axis=1, keepdims=True)
        return jnp.where(v == m, -jnp.inf, v)

    v = lax.fori_loop(0, TOPN - 1, mask_step, cand)
    theta = jnp.max(v, axis=1, keepdims=True)
    rank = jnp.where(cand >= theta, cand + cand0_ref[...], -jnp.inf)

    logz = logz_ref[...]
    vsum = jnp.zeros((BTGT, 1), jnp.float32)
    g0 = jnp.zeros((BTGT, 1), jnp.int32)
    lab_q = jnp.zeros((BTGT, 1), jnp.int32)
    for n in range(RK):
        m = jnp.max(rank, axis=1, keepdims=True)
        eq = rank == m
        pos = jnp.min(jnp.where(eq, col, BIG), axis=1, keepdims=True)
        sel = col == pos
        val = jnp.sum(jnp.where(sel, cand, 0.0), axis=1, keepdims=True)
        vsum = vsum + (val - logz)
        if n == 0:
            g0 = jnp.sum(jnp.where(sel, gcol_ref[...], 0), axis=1, keepdims=True)
            lab_q = jnp.sum(jnp.where(sel, clab_ref[...], 0), axis=1, keepdims=True)
        rank = jnp.where(sel, -jnp.inf, rank)

    lsum_ref[...] = -(jnp.sum(vsum, axis=0, keepdims=True) / (BTGT * RK))

    iota64 = lax.broadcasted_iota(jnp.int32, (BTGT, BSRC), 1)
    onehot = jnp.where(iota64 == g0, 1.0, 0.0)
    src_val = lax.dot_general(onehot, srcl_ref[...], (((1,), (0,)), ((), ())),
                              preferred_element_type=jnp.float32)
    pred = jnp.where(g0 < BSRC, src_val.astype(jnp.int32), lab_q)
    nc_ref[...] = jnp.sum(jnp.where(pred == tgt_ref[...], 1, 0),
                          axis=0, keepdims=True)


def kernel(features, target_fearures_0, source_labels, target_labels, queue,
           queue_labels):
    f32 = jnp.float32
    sim, sim0, logz, fidx, cidx = pl.pallas_call(
        _sim_body,
        grid=(NB,),
        in_specs=[
            pl.BlockSpec((BSRC + BTGT, DIM), lambda k: (0, 0)),
            pl.BlockSpec((BTGT, DIM), lambda k: (0, 0)),
            pl.BlockSpec((LBLK, DIM), lambda k: (k, 0)),
        ],
        out_specs=[
            pl.BlockSpec((BTGT, LBLK), lambda k: (0, k)),
            pl.BlockSpec((BTGT, LBLK), lambda k: (0, k)),
            pl.BlockSpec((BTGT, 1), lambda k: (0, 0)),
            pl.BlockSpec((BTGT, TOPN), lambda k: (0, 0)),
            pl.BlockSpec((BTGT, TOPN), lambda k: (0, 0)),
        ],
        out_shape=[
            jax.ShapeDtypeStruct((BTGT, KQ), f32),
            jax.ShapeDtypeStruct((BTGT, KQ), f32),
            jax.ShapeDtypeStruct((BTGT, 1), f32),
            jax.ShapeDtypeStruct((BTGT, TOPN), jnp.int32),
            jax.ShapeDtypeStruct((BTGT, TOPN), jnp.int32),
        ],
        scratch_shapes=[
            pltpu.VMEM((2 * BTGT, DIM), f32),
            pltpu.VMEM((NB, BTGT, CPB), f32),
            pltpu.VMEM((BTGT, 1), f32),
            pltpu.VMEM((BTGT, 1), f32),
        ],
    )(features, target_fearures_0, queue)

    cand, cand0, clab = _gather_sc(
        sim.reshape(BTGT * NCH, CHUNK),
        sim0.reshape(BTGT * NCH, CHUNK),
        queue_labels.reshape(NCH, CHUNK),
        fidx.reshape(BTGT * TOPN),
        cidx.reshape(BTGT * TOPN),
    )

    gcol = (cidx.reshape(BTGT, TOPN, 1) * CHUNK +
            jnp.arange(CHUNK, dtype=jnp.int32).reshape(1, 1, CHUNK)
            ).reshape(BTGT, CW)

    lsum2, nc2 = pl.pallas_call(
        _loss_body,
        out_shape=[
            jax.ShapeDtypeStruct((1, 1), f32),
            jax.ShapeDtypeStruct((1, 1), jnp.int32),
        ],
    )(
        cand.reshape(BTGT, CW),
        cand0.reshape(BTGT, CW),
        clab.reshape(BTGT, CW),
        gcol,
        logz,
        source_labels.astype(f32).reshape(BSRC, 1),
        target_labels.astype(jnp.int32).reshape(BTGT, 1),
    )
    return (lsum2[0, 0], nc2[0, 0])
